# baseline (speedup 1.0000x reference)
import functools

import jax
import jax.numpy as jnp
from jax import lax
from jax.experimental import pallas as pl
from jax.experimental.pallas import tpu as pltpu
from jax.experimental.pallas import tpu_sc as plsc

_VOCAB = 1000000
_EMB = 32
_BATCH = 16384

_NC = 2
_NS = 16
_NW = _NC * _NS
_K = 8
_L = 16


@functools.cache
def _make_gather(nbatch):
    _BPW = nbatch // _NW
    _NG = _BPW // _K
    mesh = plsc.VectorSubcoreMesh(core_axis_name="c", subcore_axis_name="s",
                                  num_cores=_NC, num_subcores=_NS)

    @functools.partial(
        pl.kernel,
        mesh=mesh,
        out_type=(
            jax.ShapeDtypeStruct((nbatch, 128), jnp.float32),
            jax.ShapeDtypeStruct((nbatch, 128), jnp.float32),
        ),
        scratch_types=[
            pltpu.VMEM((_BPW + _L,), jnp.int32),
            pltpu.VMEM((_BPW + _L,), jnp.int32),
            pltpu.VMEM((2, _K, _EMB, 128), jnp.float32),
            pltpu.VMEM((2, _K, 128), jnp.float32),
            pltpu.SemaphoreType.DMA,
            pltpu.SemaphoreType.DMA,
            pltpu.SemaphoreType.DMA,
        ],
        compiler_params=pltpu.CompilerParams(needs_layout_passes=False,
                                             disable_bounds_checks=True),
    )
    def gather_k(uid_hbm, mid_hbm, utab_hbm, mtab_hbm, uout_hbm, mout_hbm,
                 uidx_v, midx_v, stage, chunkout, sem_a, sem_b, sem_out):
        wid = lax.axis_index("s") * _NC + lax.axis_index("c")
        base = wid * _BPW
        pltpu.sync_copy(uid_hbm.at[pl.ds(base, _BPW)],
                        uidx_v.at[pl.ds(0, _BPW)])
        pltpu.sync_copy(mid_hbm.at[pl.ds(base, _BPW)],
                        midx_v.at[pl.ds(0, _BPW)])

        iota = lax.iota(jnp.int32, _L)

        def gather_one(idx_v, tab_hbm, out_hbm):
            def load_vec(g):
                return idx_v[pl.ds(pl.multiple_of(g * _K, 8), _L)]

            def issue(g, slot, sem):
                vec = load_vec(g)
                for dj in range(_K):
                    c0 = pl.multiple_of((vec[dj] >> 7) << 7, 128)
                    pltpu.async_copy(tab_hbm.at[:, pl.ds(c0, 128)],
                                     stage.at[slot, dj], sem)

            def drain_in(sem):
                for _ in range(_K):
                    pltpu.make_async_copy(tab_hbm.at[:, pl.ds(0, 128)],
                                          stage.at[0, 0], sem).wait()

            def extract_to(g, slot):
                vec = load_vec(g)
                for dj in range(_K):
                    col = jnp.full((_L,), vec[dj] & 127, jnp.int32)
                    for h in range(_EMB // _L):
                        chunkout[slot, dj, pl.ds(h * _L, _L)] = (
                            plsc.load_gather(stage.at[slot, dj],
                                             [iota + h * _L, col]))
                row0 = pl.multiple_of(base + g * _K, 8)
                pltpu.async_copy(chunkout.at[slot],
                                 out_hbm.at[pl.ds(row0, _K), :], sem_out)

            def drain_out(n):
                for _ in range(n):
                    pltpu.make_async_copy(uout_hbm.at[pl.ds(0, _K), :],
                                          chunkout.at[0], sem_out).wait()

            def two_chunks(t, with_out_drain):
                g = 2 * t
                if with_out_drain:
                    drain_out(2)
                issue_dyn = issue
                issue_dyn(g + 1, 1, sem_b)
                drain_in(sem_a)
                extract_to(g, 0)
                issue_dyn(g + 2, 0, sem_a)
                drain_in(sem_b)
                extract_to(g + 1, 1)

            issue(0, 0, sem_a)
            two_chunks(0, False)

            def body(t, carry):
                two_chunks(t, True)
                return carry

            lax.fori_loop(1, _NG // 2 - 1, body, 0, unroll=False)

            g = _NG - 2
            drain_out(2)
            issue(g + 1, 1, sem_b)
            drain_in(sem_a)
            extract_to(g, 0)
            drain_in(sem_b)
            extract_to(g + 1, 1)
            drain_out(2)

        gather_one(uidx_v, utab_hbm, uout_hbm)
        gather_one(midx_v, mtab_hbm, mout_hbm)

    return gather_k


_G = 128
_SG = 16


@functools.cache
def _make_tc_gather(tcb):
    nch = tcb // _G

    def body(uid_s, mid_s, uid2, mid2, utab, mtab, uout_ref, mout_ref,
             stage, sem_a, sem_b):
        iota_k = lax.broadcasted_iota(jnp.int32, (_SG * 128, _SG), 0)
        iota16 = lax.iota(jnp.int32, _SG)

        def run_one(idx_s, idx2, tab, out_ref):
            def issue(g, slot, sem):
                for dj in range(_G):
                    c0 = pl.multiple_of(
                        (idx_s[g * _G + dj] >> 7) << 7, 128)
                    pltpu.make_async_copy(
                        tab.at[:, pl.ds(c0, 128)],
                        stage.at[slot, :, pl.ds(dj * 128, 128)], sem).start()

            def drain(sem):
                for _ in range(_G):
                    pltpu.make_async_copy(
                        tab.at[:, pl.ds(0, 128)],
                        stage.at[0, :, pl.ds(0, 128)], sem).wait()

            def extract(g, slot):
                idxrow = idx2[g]
                pieces = []
                for s in range(_G // _SG):
                    idx16 = lax.slice_in_dim(idxrow, s * _SG, (s + 1) * _SG)
                    target = (idx16 & 127) + iota16 * 128
                    onehot = jnp.where(iota_k == target[None, :], 1.0, 0.0)
                    sub = stage[slot, :, pl.ds(s * _SG * 128, _SG * 128)]
                    pieces.append(jnp.dot(sub, onehot,
                                          preferred_element_type=jnp.float32))
                out_ref[:, pl.ds(pl.multiple_of(g * _G, 128), _G)] = (
                    jnp.concatenate(pieces, axis=1))

            issue(0, 0, sem_a)

            def loop(t, carry):
                g = 2 * t
                issue(g + 1, 1, sem_b)
                drain(sem_a)
                extract(g, 0)
                issue(g + 2, 0, sem_a)
                drain(sem_b)
                extract(g + 1, 1)
                return carry

            lax.fori_loop(0, nch // 2 - 1, loop, 0, unroll=False)
            g = nch - 2
            issue(g + 1, 1, sem_b)
            drain(sem_a)
            extract(g, 0)
            drain(sem_b)
            extract(g + 1, 1)

        run_one(uid_s, uid2, utab, uout_ref)
        run_one(mid_s, mid2, mtab, mout_ref)

    return pl.pallas_call(
        body,
        in_specs=[
            pl.BlockSpec(memory_space=pltpu.SMEM),
            pl.BlockSpec(memory_space=pltpu.SMEM),
            pl.BlockSpec((nch, _G), lambda: (0, 0)),
            pl.BlockSpec((nch, _G), lambda: (0, 0)),
            pl.BlockSpec(memory_space=pl.ANY),
            pl.BlockSpec(memory_space=pl.ANY),
        ],
        out_specs=(pl.BlockSpec((_EMB, tcb), lambda: (0, 0)),
                   pl.BlockSpec((_EMB, tcb), lambda: (0, 0))),
        out_shape=(jax.ShapeDtypeStruct((_EMB, tcb), jnp.float32),
                   jax.ShapeDtypeStruct((_EMB, tcb), jnp.float32)),
        scratch_shapes=[
            pltpu.VMEM((2, _EMB, _G * 128), jnp.float32),
            pltpu.SemaphoreType.DMA,
            pltpu.SemaphoreType.DMA,
        ],
    )


_BT = 512


def _mlp_t_body(u_ref, m_ref, w0a_ref, w0b_ref, b0_ref, w1_ref, b1_ref,
                w2_ref, b2_ref, o_ref):
    dn = (((0,), (0,)), ((), ()))
    h0 = lax.dot_general(u_ref[...], w0a_ref[...], dn,
                         preferred_element_type=jnp.float32)
    h0 += lax.dot_general(m_ref[...], w0b_ref[...], dn,
                          preferred_element_type=jnp.float32)
    h0 = jnp.maximum(h0 + b0_ref[...][None, :], 0.0)
    h1 = jnp.dot(h0, w1_ref[...], preferred_element_type=jnp.float32)
    h1 = jnp.maximum(h1 + b1_ref[...][None, :], 0.0)
    o_ref[...] = jnp.sum(h1 * w2_ref[...], axis=1) + b2_ref[...]


def _mlp_t(u_t, m_t, w0a, w0b, b0, w1, b1, w2row, b2):
    nbatch = u_t.shape[1]
    nb = nbatch // _BT
    return pl.pallas_call(
        _mlp_t_body,
        grid=(nb,),
        in_specs=[
            pl.BlockSpec((_EMB, _BT), lambda i: (0, i)),
            pl.BlockSpec((_EMB, _BT), lambda i: (0, i)),
            pl.BlockSpec((_EMB, 256), lambda i: (0, 0)),
            pl.BlockSpec((_EMB, 256), lambda i: (0, 0)),
            pl.BlockSpec((256,), lambda i: (0,)),
            pl.BlockSpec((256, 64), lambda i: (0, 0)),
            pl.BlockSpec((64,), lambda i: (0,)),
            pl.BlockSpec((1, 64), lambda i: (0, 0)),
            pl.BlockSpec((1,), lambda i: (0,)),
        ],
        out_specs=pl.BlockSpec((_BT,), lambda i: (i,)),
        out_shape=jax.ShapeDtypeStruct((nbatch,), jnp.float32),
    )(u_t, m_t, w0a, w0b, b0, w1, b1, w2row, b2)


def _mlp_body(u_ref, m_ref, w0a_ref, w0b_ref, b0_ref, w1_ref, b1_ref,
              w2_ref, b2_ref, o_ref):
    u = u_ref[:, pl.ds(0, _EMB)]
    m = m_ref[:, pl.ds(0, _EMB)]
    h0 = jnp.dot(u, w0a_ref[...], preferred_element_type=jnp.float32)
    h0 += jnp.dot(m, w0b_ref[...], preferred_element_type=jnp.float32)
    h0 = jnp.maximum(h0 + b0_ref[...][None, :], 0.0)
    h1 = jnp.dot(h0, w1_ref[...], preferred_element_type=jnp.float32)
    h1 = jnp.maximum(h1 + b1_ref[...][None, :], 0.0)
    o_ref[...] = jnp.sum(h1 * w2_ref[...], axis=1) + b2_ref[...]


def _mlp(u, m, w0a, w0b, b0, w1, b1, w2row, b2):
    nbatch = u.shape[0]
    nb = nbatch // _BT
    return pl.pallas_call(
        _mlp_body,
        grid=(nb,),
        in_specs=[
            pl.BlockSpec((_BT, 128), lambda i: (i, 0)),
            pl.BlockSpec((_BT, 128), lambda i: (i, 0)),
            pl.BlockSpec((_EMB, 256), lambda i: (0, 0)),
            pl.BlockSpec((_EMB, 256), lambda i: (0, 0)),
            pl.BlockSpec((256,), lambda i: (0,)),
            pl.BlockSpec((256, 64), lambda i: (0, 0)),
            pl.BlockSpec((64,), lambda i: (0,)),
            pl.BlockSpec((1, 64), lambda i: (0, 0)),
            pl.BlockSpec((1,), lambda i: (0,)),
        ],
        out_specs=pl.BlockSpec((_BT,), lambda i: (i,)),
        out_shape=jax.ShapeDtypeStruct((nbatch,), jnp.float32),
    )(u, m, w0a, w0b, b0, w1, b1, w2row, b2)


_SC_SPLITS = (11776,)
_SCB = sum(_SC_SPLITS)
_TCB = _BATCH - _SCB


def kernel(user_id, movie_id, user_table, movie_table, W0, b0, W1, b1, W2, b2):
    ut, mt = user_table.T, movie_table.T
    w0a, w0b, w2r = W0[:_EMB], W0[_EMB:], W2.reshape(1, 64)
    weights = (w0a, w0b, b0, W1, b1, w2r, b2)
    gathered = []
    lo = 0
    for n in _SC_SPLITS:
        gathered.append(_make_gather(n)(user_id[lo:lo + n],
                                        movie_id[lo:lo + n], ut, mt))
        lo += n
    uid_tc, mid_tc = user_id[_SCB:], movie_id[_SCB:]
    uid2 = uid_tc.reshape(_TCB // _G, _G)
    mid2 = mid_tc.reshape(_TCB // _G, _G)
    ut_tc, mt_tc = _make_tc_gather(_TCB)(uid_tc, mid_tc, uid2, mid2,
                                         ut, mt)
    out_tc = _mlp_t(ut_tc, mt_tc, *weights)
    out_sc = [_mlp(u, m, *weights) for u, m in gathered]
    return jnp.concatenate(out_sc + [out_tc])


# speedup vs baseline: 1.0353x; 1.0353x over previous
import functools

import jax
import jax.numpy as jnp
from jax import lax
from jax.experimental import pallas as pl
from jax.experimental.pallas import tpu as pltpu
from jax.experimental.pallas import tpu_sc as plsc

_VOCAB = 1000000
_EMB = 32
_BATCH = 16384

_NC = 2
_NS = 16
_NW = _NC * _NS
_K = 8
_L = 16


@functools.cache
def _make_gather(nbatch):
    _BPW = nbatch // _NW
    _NG = _BPW // _K
    mesh = plsc.VectorSubcoreMesh(core_axis_name="c", subcore_axis_name="s",
                                  num_cores=_NC, num_subcores=_NS)

    @functools.partial(
        pl.kernel,
        mesh=mesh,
        out_type=(
            jax.ShapeDtypeStruct((nbatch, 128), jnp.float32),
            jax.ShapeDtypeStruct((nbatch, 128), jnp.float32),
        ),
        scratch_types=[
            pltpu.VMEM((_BPW + _L,), jnp.int32),
            pltpu.VMEM((_BPW + _L,), jnp.int32),
            pltpu.VMEM((2, _K, _EMB, 128), jnp.float32),
            pltpu.VMEM((2, _K, 128), jnp.float32),
            pltpu.SemaphoreType.DMA,
            pltpu.SemaphoreType.DMA,
            pltpu.SemaphoreType.DMA,
        ],
        compiler_params=pltpu.CompilerParams(needs_layout_passes=False,
                                             disable_bounds_checks=True),
    )
    def gather_k(uid_hbm, mid_hbm, utab_hbm, mtab_hbm, uout_hbm, mout_hbm,
                 uidx_v, midx_v, stage, chunkout, sem_a, sem_b, sem_out):
        wid = lax.axis_index("s") * _NC + lax.axis_index("c")
        base = wid * _BPW
        pltpu.sync_copy(uid_hbm.at[pl.ds(base, _BPW)],
                        uidx_v.at[pl.ds(0, _BPW)])
        pltpu.sync_copy(mid_hbm.at[pl.ds(base, _BPW)],
                        midx_v.at[pl.ds(0, _BPW)])

        iota = lax.iota(jnp.int32, _L)

        def gather_one(idx_v, tab_hbm, out_hbm):
            def load_vec(g):
                return idx_v[pl.ds(pl.multiple_of(g * _K, 8), _L)]

            def issue(g, slot, sem):
                vec = load_vec(g)
                for dj in range(_K):
                    c0 = pl.multiple_of((vec[dj] >> 7) << 7, 128)
                    pltpu.async_copy(tab_hbm.at[:, pl.ds(c0, 128)],
                                     stage.at[slot, dj], sem)

            def drain_in(sem):
                for _ in range(_K):
                    pltpu.make_async_copy(tab_hbm.at[:, pl.ds(0, 128)],
                                          stage.at[0, 0], sem).wait()

            def extract_to(g, slot):
                vec = load_vec(g)
                for dj in range(_K):
                    col = jnp.full((_L,), vec[dj] & 127, jnp.int32)
                    for h in range(_EMB // _L):
                        chunkout[slot, dj, pl.ds(h * _L, _L)] = (
                            plsc.load_gather(stage.at[slot, dj],
                                             [iota + h * _L, col]))
                row0 = pl.multiple_of(base + g * _K, 8)
                pltpu.async_copy(chunkout.at[slot],
                                 out_hbm.at[pl.ds(row0, _K), :], sem_out)

            def drain_out(n):
                for _ in range(n):
                    pltpu.make_async_copy(uout_hbm.at[pl.ds(0, _K), :],
                                          chunkout.at[0], sem_out).wait()

            def two_chunks(t, with_out_drain):
                g = 2 * t
                if with_out_drain:
                    drain_out(2)
                issue_dyn = issue
                issue_dyn(g + 1, 1, sem_b)
                drain_in(sem_a)
                extract_to(g, 0)
                issue_dyn(g + 2, 0, sem_a)
                drain_in(sem_b)
                extract_to(g + 1, 1)

            issue(0, 0, sem_a)
            two_chunks(0, False)

            def body(t, carry):
                two_chunks(t, True)
                return carry

            lax.fori_loop(1, _NG // 2 - 1, body, 0, unroll=False)

            g = _NG - 2
            drain_out(2)
            issue(g + 1, 1, sem_b)
            drain_in(sem_a)
            extract_to(g, 0)
            drain_in(sem_b)
            extract_to(g + 1, 1)
            drain_out(2)

        gather_one(uidx_v, utab_hbm, uout_hbm)
        gather_one(midx_v, mtab_hbm, mout_hbm)

    return gather_k


_G = 128
_SG = 16


@functools.cache
def _make_tc_gather(tcb):
    nch = tcb // _G

    def body(uid_s, mid_s, uid2, mid2, utab, mtab, uout_ref, mout_ref,
             stage, sem_a, sem_b):
        iota_k = lax.broadcasted_iota(jnp.int32, (_SG * 128, _SG), 0)
        iota16 = lax.iota(jnp.int32, _SG)

        def run_one(idx_s, idx2, tab, out_ref):
            def issue(g, slot, sem):
                for dj in range(_G):
                    c0 = pl.multiple_of(
                        (idx_s[g * _G + dj] >> 7) << 7, 128)
                    pltpu.make_async_copy(
                        tab.at[:, pl.ds(c0, 128)],
                        stage.at[slot, :, pl.ds(dj * 128, 128)], sem).start()

            def drain(sem):
                for _ in range(_G):
                    pltpu.make_async_copy(
                        tab.at[:, pl.ds(0, 128)],
                        stage.at[0, :, pl.ds(0, 128)], sem).wait()

            def extract(g, slot):
                idxrow = idx2[g]
                pieces = []
                for s in range(_G // _SG):
                    idx16 = lax.slice_in_dim(idxrow, s * _SG, (s + 1) * _SG)
                    target = (idx16 & 127) + iota16 * 128
                    onehot = jnp.where(iota_k == target[None, :], 1.0, 0.0)
                    sub = stage[slot, :, pl.ds(s * _SG * 128, _SG * 128)]
                    pieces.append(jnp.dot(sub, onehot,
                                          preferred_element_type=jnp.float32))
                out_ref[:, pl.ds(pl.multiple_of(g * _G, 128), _G)] = (
                    jnp.concatenate(pieces, axis=1))

            issue(0, 0, sem_a)

            def loop(t, carry):
                g = 2 * t
                issue(g + 1, 1, sem_b)
                drain(sem_a)
                extract(g, 0)
                issue(g + 2, 0, sem_a)
                drain(sem_b)
                extract(g + 1, 1)
                return carry

            lax.fori_loop(0, nch // 2 - 1, loop, 0, unroll=False)
            g = nch - 2
            issue(g + 1, 1, sem_b)
            drain(sem_a)
            extract(g, 0)
            drain(sem_b)
            extract(g + 1, 1)

        run_one(uid_s, uid2, utab, uout_ref)
        run_one(mid_s, mid2, mtab, mout_ref)

    return pl.pallas_call(
        body,
        in_specs=[
            pl.BlockSpec(memory_space=pltpu.SMEM),
            pl.BlockSpec(memory_space=pltpu.SMEM),
            pl.BlockSpec((nch, _G), lambda: (0, 0)),
            pl.BlockSpec((nch, _G), lambda: (0, 0)),
            pl.BlockSpec(memory_space=pl.ANY),
            pl.BlockSpec(memory_space=pl.ANY),
        ],
        out_specs=(pl.BlockSpec((_EMB, tcb), lambda: (0, 0)),
                   pl.BlockSpec((_EMB, tcb), lambda: (0, 0))),
        out_shape=(jax.ShapeDtypeStruct((_EMB, tcb), jnp.float32),
                   jax.ShapeDtypeStruct((_EMB, tcb), jnp.float32)),
        scratch_shapes=[
            pltpu.VMEM((2, _EMB, _G * 128), jnp.float32),
            pltpu.SemaphoreType.DMA,
            pltpu.SemaphoreType.DMA,
        ],
    )


_BT = 1024


def _mlp_t_body(u_ref, m_ref, w0a_ref, w0b_ref, b0_ref, w1_ref, b1_ref,
                w2_ref, b2_ref, o_ref):
    dn = (((0,), (0,)), ((), ()))
    h0 = lax.dot_general(u_ref[...], w0a_ref[...], dn,
                         preferred_element_type=jnp.float32)
    h0 += lax.dot_general(m_ref[...], w0b_ref[...], dn,
                          preferred_element_type=jnp.float32)
    h0 = jnp.maximum(h0 + b0_ref[...][None, :], 0.0)
    h1 = jnp.dot(h0, w1_ref[...], preferred_element_type=jnp.float32)
    h1 = jnp.maximum(h1 + b1_ref[...][None, :], 0.0)
    o_ref[...] = jnp.sum(h1 * w2_ref[...], axis=1) + b2_ref[...]


def _mlp_t(u_t, m_t, w0a, w0b, b0, w1, b1, w2row, b2):
    nbatch = u_t.shape[1]
    nb = nbatch // _BT
    return pl.pallas_call(
        _mlp_t_body,
        grid=(nb,),
        in_specs=[
            pl.BlockSpec((_EMB, _BT), lambda i: (0, i)),
            pl.BlockSpec((_EMB, _BT), lambda i: (0, i)),
            pl.BlockSpec((_EMB, 256), lambda i: (0, 0)),
            pl.BlockSpec((_EMB, 256), lambda i: (0, 0)),
            pl.BlockSpec((256,), lambda i: (0,)),
            pl.BlockSpec((256, 64), lambda i: (0, 0)),
            pl.BlockSpec((64,), lambda i: (0,)),
            pl.BlockSpec((1, 64), lambda i: (0, 0)),
            pl.BlockSpec((1,), lambda i: (0,)),
        ],
        out_specs=pl.BlockSpec((_BT,), lambda i: (i,)),
        out_shape=jax.ShapeDtypeStruct((nbatch,), jnp.float32),
    )(u_t, m_t, w0a, w0b, b0, w1, b1, w2row, b2)


def _mlp_body(u_ref, m_ref, w0a_ref, w0b_ref, b0_ref, w1_ref, b1_ref,
              w2_ref, b2_ref, o_ref):
    u = u_ref[:, pl.ds(0, _EMB)]
    m = m_ref[:, pl.ds(0, _EMB)]
    h0 = jnp.dot(u, w0a_ref[...], preferred_element_type=jnp.float32)
    h0 += jnp.dot(m, w0b_ref[...], preferred_element_type=jnp.float32)
    h0 = jnp.maximum(h0 + b0_ref[...][None, :], 0.0)
    h1 = jnp.dot(h0, w1_ref[...], preferred_element_type=jnp.float32)
    h1 = jnp.maximum(h1 + b1_ref[...][None, :], 0.0)
    o_ref[...] = jnp.sum(h1 * w2_ref[...], axis=1) + b2_ref[...]


def _mlp(u, m, w0a, w0b, b0, w1, b1, w2row, b2):
    nbatch = u.shape[0]
    nb = nbatch // _BT
    return pl.pallas_call(
        _mlp_body,
        grid=(nb,),
        in_specs=[
            pl.BlockSpec((_BT, 128), lambda i: (i, 0)),
            pl.BlockSpec((_BT, 128), lambda i: (i, 0)),
            pl.BlockSpec((_EMB, 256), lambda i: (0, 0)),
            pl.BlockSpec((_EMB, 256), lambda i: (0, 0)),
            pl.BlockSpec((256,), lambda i: (0,)),
            pl.BlockSpec((256, 64), lambda i: (0, 0)),
            pl.BlockSpec((64,), lambda i: (0,)),
            pl.BlockSpec((1, 64), lambda i: (0, 0)),
            pl.BlockSpec((1,), lambda i: (0,)),
        ],
        out_specs=pl.BlockSpec((_BT,), lambda i: (i,)),
        out_shape=jax.ShapeDtypeStruct((nbatch,), jnp.float32),
    )(u, m, w0a, w0b, b0, w1, b1, w2row, b2)


_SC_SPLITS = (11264,)
_SCB = sum(_SC_SPLITS)
_TCB = _BATCH - _SCB


def kernel(user_id, movie_id, user_table, movie_table, W0, b0, W1, b1, W2, b2):
    ut, mt = user_table.T, movie_table.T
    w0a, w0b, w2r = W0[:_EMB], W0[_EMB:], W2.reshape(1, 64)
    weights = (w0a, w0b, b0, W1, b1, w2r, b2)
    gathered = []
    lo = 0
    for n in _SC_SPLITS:
        gathered.append(_make_gather(n)(user_id[lo:lo + n],
                                        movie_id[lo:lo + n], ut, mt))
        lo += n
    uid_tc, mid_tc = user_id[_SCB:], movie_id[_SCB:]
    uid2 = uid_tc.reshape(_TCB // _G, _G)
    mid2 = mid_tc.reshape(_TCB // _G, _G)
    ut_tc, mt_tc = _make_tc_gather(_TCB)(uid_tc, mid_tc, uid2, mid2,
                                         ut, mt)
    out_tc = _mlp_t(ut_tc, mt_tc, *weights)
    out_sc = [_mlp(u, m, *weights) for u, m in gathered]
    return jnp.concatenate(out_sc + [out_tc])


# speedup vs baseline: 1.0376x; 1.0022x over previous
import functools

import jax
import jax.numpy as jnp
from jax import lax
from jax.experimental import pallas as pl
from jax.experimental.pallas import tpu as pltpu
from jax.experimental.pallas import tpu_sc as plsc

_VOCAB = 1000000
_EMB = 32
_BATCH = 16384

_NC = 2
_NS = 16
_NW = _NC * _NS
_K = 8
_L = 16


@functools.cache
def _make_gather(nbatch):
    _BPW = nbatch // _NW
    _NG = _BPW // _K
    mesh = plsc.VectorSubcoreMesh(core_axis_name="c", subcore_axis_name="s",
                                  num_cores=_NC, num_subcores=_NS)

    @functools.partial(
        pl.kernel,
        mesh=mesh,
        out_type=(
            jax.ShapeDtypeStruct((nbatch, 128), jnp.float32),
            jax.ShapeDtypeStruct((nbatch, 128), jnp.float32),
        ),
        scratch_types=[
            pltpu.VMEM((_BPW + _L,), jnp.int32),
            pltpu.VMEM((_BPW + _L,), jnp.int32),
            pltpu.VMEM((2, _K, _EMB, 128), jnp.float32),
            pltpu.VMEM((2, _K, 128), jnp.float32),
            pltpu.SemaphoreType.DMA,
            pltpu.SemaphoreType.DMA,
            pltpu.SemaphoreType.DMA,
        ],
        compiler_params=pltpu.CompilerParams(needs_layout_passes=False,
                                             disable_bounds_checks=True),
    )
    def gather_k(uid_hbm, mid_hbm, utab_hbm, mtab_hbm, uout_hbm, mout_hbm,
                 uidx_v, midx_v, stage, chunkout, sem_a, sem_b, sem_out):
        wid = lax.axis_index("s") * _NC + lax.axis_index("c")
        base = wid * _BPW
        pltpu.sync_copy(uid_hbm.at[pl.ds(base, _BPW)],
                        uidx_v.at[pl.ds(0, _BPW)])
        pltpu.sync_copy(mid_hbm.at[pl.ds(base, _BPW)],
                        midx_v.at[pl.ds(0, _BPW)])

        iota = lax.iota(jnp.int32, _L)

        def gather_one(idx_v, tab_hbm, out_hbm):
            def load_vec(g):
                return idx_v[pl.ds(pl.multiple_of(g * _K, 8), _L)]

            def issue(g, slot, sem):
                vec = load_vec(g)
                for dj in range(_K):
                    c0 = pl.multiple_of((vec[dj] >> 7) << 7, 128)
                    pltpu.async_copy(tab_hbm.at[:, pl.ds(c0, 128)],
                                     stage.at[slot, dj], sem)

            def drain_in(sem):
                for _ in range(_K):
                    pltpu.make_async_copy(tab_hbm.at[:, pl.ds(0, 128)],
                                          stage.at[0, 0], sem).wait()

            def extract_to(g, slot):
                vec = load_vec(g)
                for dj in range(_K):
                    col = jnp.full((_L,), vec[dj] & 127, jnp.int32)
                    for h in range(_EMB // _L):
                        chunkout[slot, dj, pl.ds(h * _L, _L)] = (
                            plsc.load_gather(stage.at[slot, dj],
                                             [iota + h * _L, col]))
                row0 = pl.multiple_of(base + g * _K, 8)
                pltpu.async_copy(chunkout.at[slot],
                                 out_hbm.at[pl.ds(row0, _K), :], sem_out)

            def drain_out(n):
                for _ in range(n):
                    pltpu.make_async_copy(uout_hbm.at[pl.ds(0, _K), :],
                                          chunkout.at[0], sem_out).wait()

            def two_chunks(t, with_out_drain):
                g = 2 * t
                if with_out_drain:
                    drain_out(2)
                issue_dyn = issue
                issue_dyn(g + 1, 1, sem_b)
                drain_in(sem_a)
                extract_to(g, 0)
                issue_dyn(g + 2, 0, sem_a)
                drain_in(sem_b)
                extract_to(g + 1, 1)

            issue(0, 0, sem_a)
            two_chunks(0, False)

            def body(t, carry):
                two_chunks(t, True)
                return carry

            lax.fori_loop(1, _NG // 2 - 1, body, 0, unroll=False)

            g = _NG - 2
            drain_out(2)
            issue(g + 1, 1, sem_b)
            drain_in(sem_a)
            extract_to(g, 0)
            drain_in(sem_b)
            extract_to(g + 1, 1)
            drain_out(2)

        gather_one(uidx_v, utab_hbm, uout_hbm)
        gather_one(midx_v, mtab_hbm, mout_hbm)

    return gather_k


_G = 128
_SG = 16


@functools.cache
def _make_tc_gather(tcb):
    nch = tcb // _G

    def body(uid_s, mid_s, uid2, mid2, utab, mtab, uout_ref, mout_ref,
             stage, sem_a, sem_b):
        iota_k = lax.broadcasted_iota(jnp.int32, (_SG * 128, _SG), 0)
        iota16 = lax.iota(jnp.int32, _SG)

        def run_one(idx_s, idx2, tab, out_ref):
            def issue(g, slot, sem):
                for dj in range(_G):
                    c0 = pl.multiple_of(
                        (idx_s[g * _G + dj] >> 7) << 7, 128)
                    pltpu.make_async_copy(
                        tab.at[:, pl.ds(c0, 128)],
                        stage.at[slot, :, pl.ds(dj * 128, 128)], sem).start()

            def drain(sem):
                for _ in range(_G):
                    pltpu.make_async_copy(
                        tab.at[:, pl.ds(0, 128)],
                        stage.at[0, :, pl.ds(0, 128)], sem).wait()

            def extract(g, slot):
                idxrow = idx2[g]
                pieces = []
                for s in range(_G // _SG):
                    idx16 = lax.slice_in_dim(idxrow, s * _SG, (s + 1) * _SG)
                    target = (idx16 & 127) + iota16 * 128
                    onehot = jnp.where(iota_k == target[None, :],
                                       1.0, 0.0).astype(jnp.bfloat16)
                    sub = stage[slot, :, pl.ds(s * _SG * 128, _SG * 128)]
                    pieces.append(jnp.dot(sub.astype(jnp.bfloat16), onehot,
                                          preferred_element_type=jnp.float32))
                out_ref[:, pl.ds(pl.multiple_of(g * _G, 128), _G)] = (
                    jnp.concatenate(pieces, axis=1))

            issue(0, 0, sem_a)

            def loop(t, carry):
                g = 2 * t
                issue(g + 1, 1, sem_b)
                drain(sem_a)
                extract(g, 0)
                issue(g + 2, 0, sem_a)
                drain(sem_b)
                extract(g + 1, 1)
                return carry

            lax.fori_loop(0, nch // 2 - 1, loop, 0, unroll=False)
            g = nch - 2
            issue(g + 1, 1, sem_b)
            drain(sem_a)
            extract(g, 0)
            drain(sem_b)
            extract(g + 1, 1)

        run_one(uid_s, uid2, utab, uout_ref)
        run_one(mid_s, mid2, mtab, mout_ref)

    return pl.pallas_call(
        body,
        in_specs=[
            pl.BlockSpec(memory_space=pltpu.SMEM),
            pl.BlockSpec(memory_space=pltpu.SMEM),
            pl.BlockSpec((nch, _G), lambda: (0, 0)),
            pl.BlockSpec((nch, _G), lambda: (0, 0)),
            pl.BlockSpec(memory_space=pl.ANY),
            pl.BlockSpec(memory_space=pl.ANY),
        ],
        out_specs=(pl.BlockSpec((_EMB, tcb), lambda: (0, 0)),
                   pl.BlockSpec((_EMB, tcb), lambda: (0, 0))),
        out_shape=(jax.ShapeDtypeStruct((_EMB, tcb), jnp.float32),
                   jax.ShapeDtypeStruct((_EMB, tcb), jnp.float32)),
        scratch_shapes=[
            pltpu.VMEM((2, _EMB, _G * 128), jnp.float32),
            pltpu.SemaphoreType.DMA,
            pltpu.SemaphoreType.DMA,
        ],
    )


_BT = 1024


def _mlp_t_body(u_ref, m_ref, w0a_ref, w0b_ref, b0_ref, w1_ref, b1_ref,
                w2_ref, b2_ref, o_ref):
    dn = (((0,), (0,)), ((), ()))
    h0 = lax.dot_general(u_ref[...], w0a_ref[...], dn,
                         preferred_element_type=jnp.float32)
    h0 += lax.dot_general(m_ref[...], w0b_ref[...], dn,
                          preferred_element_type=jnp.float32)
    h0 = jnp.maximum(h0 + b0_ref[...][None, :], 0.0)
    h1 = jnp.dot(h0, w1_ref[...], preferred_element_type=jnp.float32)
    h1 = jnp.maximum(h1 + b1_ref[...][None, :], 0.0)
    o_ref[...] = jnp.sum(h1 * w2_ref[...], axis=1) + b2_ref[...]


def _mlp_t(u_t, m_t, w0a, w0b, b0, w1, b1, w2row, b2):
    nbatch = u_t.shape[1]
    nb = nbatch // _BT
    return pl.pallas_call(
        _mlp_t_body,
        grid=(nb,),
        in_specs=[
            pl.BlockSpec((_EMB, _BT), lambda i: (0, i)),
            pl.BlockSpec((_EMB, _BT), lambda i: (0, i)),
            pl.BlockSpec((_EMB, 256), lambda i: (0, 0)),
            pl.BlockSpec((_EMB, 256), lambda i: (0, 0)),
            pl.BlockSpec((256,), lambda i: (0,)),
            pl.BlockSpec((256, 64), lambda i: (0, 0)),
            pl.BlockSpec((64,), lambda i: (0,)),
            pl.BlockSpec((1, 64), lambda i: (0, 0)),
            pl.BlockSpec((1,), lambda i: (0,)),
        ],
        out_specs=pl.BlockSpec((_BT,), lambda i: (i,)),
        out_shape=jax.ShapeDtypeStruct((nbatch,), jnp.float32),
    )(u_t, m_t, w0a, w0b, b0, w1, b1, w2row, b2)


def _mlp_body(u_ref, m_ref, w0a_ref, w0b_ref, b0_ref, w1_ref, b1_ref,
              w2_ref, b2_ref, o_ref):
    u = u_ref[:, pl.ds(0, _EMB)]
    m = m_ref[:, pl.ds(0, _EMB)]
    h0 = jnp.dot(u, w0a_ref[...], preferred_element_type=jnp.float32)
    h0 += jnp.dot(m, w0b_ref[...], preferred_element_type=jnp.float32)
    h0 = jnp.maximum(h0 + b0_ref[...][None, :], 0.0)
    h1 = jnp.dot(h0, w1_ref[...], preferred_element_type=jnp.float32)
    h1 = jnp.maximum(h1 + b1_ref[...][None, :], 0.0)
    o_ref[...] = jnp.sum(h1 * w2_ref[...], axis=1) + b2_ref[...]


def _mlp(u, m, w0a, w0b, b0, w1, b1, w2row, b2):
    nbatch = u.shape[0]
    nb = nbatch // _BT
    return pl.pallas_call(
        _mlp_body,
        grid=(nb,),
        in_specs=[
            pl.BlockSpec((_BT, 128), lambda i: (i, 0)),
            pl.BlockSpec((_BT, 128), lambda i: (i, 0)),
            pl.BlockSpec((_EMB, 256), lambda i: (0, 0)),
            pl.BlockSpec((_EMB, 256), lambda i: (0, 0)),
            pl.BlockSpec((256,), lambda i: (0,)),
            pl.BlockSpec((256, 64), lambda i: (0, 0)),
            pl.BlockSpec((64,), lambda i: (0,)),
            pl.BlockSpec((1, 64), lambda i: (0, 0)),
            pl.BlockSpec((1,), lambda i: (0,)),
        ],
        out_specs=pl.BlockSpec((_BT,), lambda i: (i,)),
        out_shape=jax.ShapeDtypeStruct((nbatch,), jnp.float32),
    )(u, m, w0a, w0b, b0, w1, b1, w2row, b2)


_SC_SPLITS = (11264,)
_SCB = sum(_SC_SPLITS)
_TCB = _BATCH - _SCB


def kernel(user_id, movie_id, user_table, movie_table, W0, b0, W1, b1, W2, b2):
    ut, mt = user_table.T, movie_table.T
    w0a, w0b, w2r = W0[:_EMB], W0[_EMB:], W2.reshape(1, 64)
    weights = (w0a, w0b, b0, W1, b1, w2r, b2)
    gathered = []
    lo = 0
    for n in _SC_SPLITS:
        gathered.append(_make_gather(n)(user_id[lo:lo + n],
                                        movie_id[lo:lo + n], ut, mt))
        lo += n
    uid_tc, mid_tc = user_id[_SCB:], movie_id[_SCB:]
    uid2 = uid_tc.reshape(_TCB // _G, _G)
    mid2 = mid_tc.reshape(_TCB // _G, _G)
    ut_tc, mt_tc = _make_tc_gather(_TCB)(uid_tc, mid_tc, uid2, mid2,
                                         ut, mt)
    out_tc = _mlp_t(ut_tc, mt_tc, *weights)
    out_sc = [_mlp(u, m, *weights) for u, m in gathered]
    return jnp.concatenate(out_sc + [out_tc])
